# baseline (speedup 1.0000x reference)
import functools

import jax
import jax.numpy as jnp
from jax.experimental import pallas as pl

_K = 10
_R = 512
_SWEEPS = 6


def _cov_body(ptsq_ref, p_ref, qpack_ref, out_ref):
    q8 = ptsq_ref[0]
    p = p_ref[0]
    qpack = qpack_ref[0]

    pq = jax.lax.dot_general(
        p, q8, (((1,), (0,)), ((), ())),
        preferred_element_type=jnp.float32)
    scores = q8[3:4, :] - 2.0 * pq

    m = jnp.min(scores, axis=1, keepdims=True)
    for _ in range(_K - 1):
        m = jnp.min(jnp.where(scores > m, scores, jnp.inf),
                    axis=1, keepdims=True)
    msum = jnp.where(scores <= m, 1.0, 0.0)

    mb = msum.astype(jnp.bfloat16)
    spack = jax.lax.dot_general(
        qpack, mb, (((0,), (1,)), ((), ())),
        preferred_element_type=jnp.float32)
    s12 = spack[0:16, :] + (spack[16:32, :] + spack[32:48, :])

    inv_k = 1.0 / _K
    mux = s12[0:1, :] * inv_k
    muy = s12[1:2, :] * inv_k
    muz = s12[2:3, :] * inv_k
    cxx = s12[3:4, :] * inv_k - mux * mux
    cyy = s12[4:5, :] * inv_k - muy * muy
    czz = s12[5:6, :] * inv_k - muz * muz
    cxy = s12[6:7, :] * inv_k - mux * muy
    cxz = s12[7:8, :] * inv_k - mux * muz
    cyz = s12[8:9, :] * inv_k - muy * muz
    z0 = jnp.zeros_like(cxx)
    out_ref[0] = jnp.concatenate([cxx, cyy, czz, cxy, cxz, cyz, z0, z0],
                                 axis=0)


def _hk(i, j):
    return (min(i, j), max(i, j))


def _jacobi_rotate(h, v, p, q):
    r = 3 - p - q
    a = h[_hk(p, p)]
    b = h[_hk(q, q)]
    d = h[_hk(p, q)]
    e = h[_hk(p, r)]
    f = h[_hk(q, r)]
    tau = (b - a) / (2.0 * d)
    t = jnp.sign(tau) / (jnp.abs(tau) + jnp.sqrt(1.0 + tau * tau))
    t = jnp.where(d == 0.0, 0.0, t)
    c = jax.lax.rsqrt(1.0 + t * t)
    s = t * c
    cc = c * c
    ss = s * s
    sc2 = 2.0 * s * c
    h[_hk(p, p)] = cc * a - sc2 * d + ss * b
    h[_hk(q, q)] = ss * a + sc2 * d + cc * b
    h[_hk(p, q)] = s * c * (a - b) + (cc - ss) * d
    h[_hk(p, r)] = c * e - s * f
    h[_hk(q, r)] = s * e + c * f
    for i in range(3):
        vp = v[(i, p)]
        vq = v[(i, q)]
        v[(i, p)] = c * vp - s * vq
        v[(i, q)] = s * vp + c * vq


def _eig_mse_body(cov_ref, out_ref):
    h = {}
    h[(0, 0)] = cov_ref[0]
    h[(1, 1)] = cov_ref[1]
    h[(2, 2)] = cov_ref[2]
    h[(0, 1)] = cov_ref[3]
    h[(0, 2)] = cov_ref[4]
    h[(1, 2)] = cov_ref[5]
    one = jnp.ones_like(h[(0, 0)])
    zero = jnp.zeros_like(one)
    v = {}
    for i in range(3):
        for j in range(3):
            v[(i, j)] = one if i == j else zero

    for _ in range(_SWEEPS):
        _jacobi_rotate(h, v, 0, 2)
        _jacobi_rotate(h, v, 2, 1)
        _jacobi_rotate(h, v, 0, 1)

    w0, w1, w2 = h[(0, 0)], h[(1, 1)], h[(2, 2)]
    sel0 = jnp.logical_and(w0 <= w1, w0 <= w2)
    sel1 = jnp.logical_and(w1 < w0, w1 <= w2)
    n = []
    for i in range(3):
        n.append(jnp.where(sel0, v[(i, 0)],
                           jnp.where(sel1, v[(i, 1)], v[(i, 2)])))

    acc = jnp.zeros_like(n[0][:8, :])
    for i in range(3):
        dlt = n[i][:8, :] - n[i][8:, :]
        acc = acc + dlt * dlt
    out_ref[...] = jnp.zeros((8, 128), jnp.float32) + jnp.sum(acc)


@jax.jit
def _normals_mse(pts):
    g, _, n = pts.shape
    x = pts[:, 0]
    y = pts[:, 1]
    z = pts[:, 2]
    sq = x * x + y * y + z * z
    zn = jnp.zeros((g, n), dtype=pts.dtype)
    ptsq = jnp.stack([x, y, z, sq, zn, zn, zn, zn], axis=1)
    pts_n8 = jnp.transpose(
        jnp.stack([x, y, z, zn, zn, zn, zn, zn], axis=1), (0, 2, 1))
    q12 = jnp.stack(
        [x, y, z, x * x, y * y, z * z, x * y, x * z, y * z,
         zn, zn, zn, zn, zn, zn, zn], axis=1)
    q12 = jnp.transpose(q12, (0, 2, 1))
    qhi = q12.astype(jnp.bfloat16)
    qrem = q12 - qhi.astype(jnp.float32)
    qmid = qrem.astype(jnp.bfloat16)
    qlo = (qrem - qmid.astype(jnp.float32)).astype(jnp.bfloat16)
    qpack = jnp.concatenate([qhi, qmid, qlo], axis=2)

    covs = pl.pallas_call(
        _cov_body,
        grid=(g, n // _R),
        in_specs=[
            pl.BlockSpec((1, 8, n), lambda b, i: (b, 0, 0)),
            pl.BlockSpec((1, _R, 8), lambda b, i: (b, i, 0)),
            pl.BlockSpec((1, n, 48), lambda b, i: (b, 0, 0)),
        ],
        out_specs=pl.BlockSpec((1, 8, _R), lambda b, i: (b, 0, i)),
        out_shape=jax.ShapeDtypeStruct((g, 8, n), jnp.float32),
    )(ptsq, pts_n8, qpack)

    total = g * n
    cov6 = jnp.transpose(covs, (1, 0, 2)).reshape(8, 16, total // 16)[:6]
    sq_sum = pl.pallas_call(
        _eig_mse_body,
        out_shape=jax.ShapeDtypeStruct((8, 128), jnp.float32),
    )(cov6)
    return sq_sum[0, 0]


def kernel(pred, gt):
    b, c, n = pred.shape
    pts = jnp.concatenate([pred, gt], axis=0)
    return _normals_mse(pts) / (b * c * n)


# speedup vs baseline: 1.0474x; 1.0474x over previous
import functools

import jax
import jax.numpy as jnp
from jax.experimental import pallas as pl

_K = 10
_R = 512
_SWEEPS = 6


def _cov_body(ptsq_ref, p_ref, qpack_ref, out_ref):
    q8 = ptsq_ref[0]
    p3 = p_ref[0, 0:3, :]
    qpack = qpack_ref[0]

    pq = jax.lax.dot_general(
        p3, q8[0:3, :], (((0,), (0,)), ((), ())),
        preferred_element_type=jnp.float32)
    scores = q8[3:4, :] - 2.0 * pq

    m = jnp.min(scores, axis=1, keepdims=True)
    for _ in range(_K - 1):
        m = jnp.min(jnp.where(scores > m, scores, jnp.inf),
                    axis=1, keepdims=True)
    msum = jnp.where(scores <= m, 1.0, 0.0)

    mb = msum.astype(jnp.bfloat16)
    spack = jax.lax.dot_general(
        qpack, mb, (((1,), (1,)), ((), ())),
        preferred_element_type=jnp.float32)
    s12 = spack[0:16, :] + (spack[16:32, :] + spack[32:48, :])

    inv_k = 1.0 / _K
    mux = s12[0:1, :] * inv_k
    muy = s12[1:2, :] * inv_k
    muz = s12[2:3, :] * inv_k
    cxx = s12[3:4, :] * inv_k - mux * mux
    cyy = s12[4:5, :] * inv_k - muy * muy
    czz = s12[5:6, :] * inv_k - muz * muz
    cxy = s12[6:7, :] * inv_k - mux * muy
    cxz = s12[7:8, :] * inv_k - mux * muz
    cyz = s12[8:9, :] * inv_k - muy * muz
    z0 = jnp.zeros_like(cxx)
    out_ref[...] = jnp.concatenate([cxx, cyy, czz, cxy, cxz, cyz, z0, z0],
                                   axis=0)


def _hk(i, j):
    return (min(i, j), max(i, j))


def _jacobi_rotate(h, v, p, q):
    r = 3 - p - q
    a = h[_hk(p, p)]
    b = h[_hk(q, q)]
    d = h[_hk(p, q)]
    e = h[_hk(p, r)]
    f = h[_hk(q, r)]
    tau = (b - a) / (2.0 * d)
    t = jnp.sign(tau) / (jnp.abs(tau) + jnp.sqrt(1.0 + tau * tau))
    t = jnp.where(d == 0.0, 0.0, t)
    c = jax.lax.rsqrt(1.0 + t * t)
    s = t * c
    cc = c * c
    ss = s * s
    sc2 = 2.0 * s * c
    h[_hk(p, p)] = cc * a - sc2 * d + ss * b
    h[_hk(q, q)] = ss * a + sc2 * d + cc * b
    h[_hk(p, q)] = s * c * (a - b) + (cc - ss) * d
    h[_hk(p, r)] = c * e - s * f
    h[_hk(q, r)] = s * e + c * f
    for i in range(3):
        vp = v[(i, p)]
        vq = v[(i, q)]
        v[(i, p)] = c * vp - s * vq
        v[(i, q)] = s * vp + c * vq


def _eig_mse_body(cov_ref, out_ref):
    h = {}
    h[(0, 0)] = cov_ref[0]
    h[(1, 1)] = cov_ref[1]
    h[(2, 2)] = cov_ref[2]
    h[(0, 1)] = cov_ref[3]
    h[(0, 2)] = cov_ref[4]
    h[(1, 2)] = cov_ref[5]
    one = jnp.ones_like(h[(0, 0)])
    zero = jnp.zeros_like(one)
    v = {}
    for i in range(3):
        for j in range(3):
            v[(i, j)] = one if i == j else zero

    for _ in range(_SWEEPS):
        _jacobi_rotate(h, v, 0, 2)
        _jacobi_rotate(h, v, 2, 1)
        _jacobi_rotate(h, v, 0, 1)

    w0, w1, w2 = h[(0, 0)], h[(1, 1)], h[(2, 2)]
    sel0 = jnp.logical_and(w0 <= w1, w0 <= w2)
    sel1 = jnp.logical_and(w1 < w0, w1 <= w2)
    n = []
    for i in range(3):
        n.append(jnp.where(sel0, v[(i, 0)],
                           jnp.where(sel1, v[(i, 1)], v[(i, 2)])))

    acc = jnp.zeros_like(n[0][:8, :])
    for i in range(3):
        dlt = n[i][:8, :] - n[i][8:, :]
        acc = acc + dlt * dlt
    out_ref[...] = jnp.zeros((8, 128), jnp.float32) + jnp.sum(acc)


@jax.jit
def _normals_mse(pts):
    g, _, n = pts.shape
    x = pts[:, 0]
    y = pts[:, 1]
    z = pts[:, 2]
    sq = x * x + y * y + z * z
    zn = jnp.zeros((g, n), dtype=pts.dtype)
    ptsq = jnp.stack([x, y, z, sq, zn, zn, zn, zn], axis=1)
    q12 = jnp.stack(
        [x, y, z, x * x, y * y, z * z, x * y, x * z, y * z,
         zn, zn, zn, zn, zn, zn, zn], axis=1)
    qhi = q12.astype(jnp.bfloat16)
    qrem = q12 - qhi.astype(jnp.float32)
    qmid = qrem.astype(jnp.bfloat16)
    qlo = (qrem - qmid.astype(jnp.float32)).astype(jnp.bfloat16)
    qpack = jnp.concatenate([qhi, qmid, qlo], axis=1)

    nblk = n // _R
    covs = pl.pallas_call(
        _cov_body,
        grid=(g, nblk),
        in_specs=[
            pl.BlockSpec((1, 8, n), lambda b, i: (b, 0, 0)),
            pl.BlockSpec((1, 8, _R), lambda b, i: (b, 0, i)),
            pl.BlockSpec((1, 48, n), lambda b, i: (b, 0, 0)),
        ],
        out_specs=pl.BlockSpec((8, _R), lambda b, i: (0, b * nblk + i)),
        out_shape=jax.ShapeDtypeStruct((8, g * n), jnp.float32),
    )(ptsq, ptsq, qpack)

    total = g * n
    cov8 = covs.reshape(8, 16, total // 16)
    sq_sum = pl.pallas_call(
        _eig_mse_body,
        out_shape=jax.ShapeDtypeStruct((8, 128), jnp.float32),
    )(cov8)
    return sq_sum[0, 0]


def kernel(pred, gt):
    b, c, n = pred.shape
    pts = jnp.concatenate([pred, gt], axis=0)
    return _normals_mse(pts) / (b * c * n)
